# baseline (speedup 1.0000x reference)
import functools

import jax
import jax.numpy as jnp
from jax import lax
from jax.experimental import pallas as pl
from jax.experimental.pallas import tpu as pltpu
from jax.experimental.pallas import tpu_sc as plsc

N = 10000
D = 128
H = 128
G = 64
E = 320000

NC, NS, L = 2, 16, 16
NW = NC * NS
S_TILE = 10240
EPAD = NW * S_TILE
NCHUNK = S_TILE // 128
N_PAD = 10240
ROWS_PER_TILE = N_PAD // NS
HH = H // 2


def _leaky(x, slope):
    return jnp.where(x >= 0, x, slope * x)


def _proj_body(x_ref, w_ref, a_ref, h_ref, av_ref):
    h = jnp.dot(x_ref[...], w_ref[...], preferred_element_type=jnp.float32)
    h_ref[...] = h
    av_ref[...] = jnp.dot(h, a_ref[...], preferred_element_type=jnp.float32)


def _projections(x, W, a_src, a_dst, rows_per_block=2000):
    nb = N // rows_per_block
    a2 = jnp.stack([a_src, a_dst], axis=1)
    h, av = pl.pallas_call(
        _proj_body,
        grid=(nb,),
        in_specs=[
            pl.BlockSpec((rows_per_block, D), lambda i: (i, 0)),
            pl.BlockSpec((D, H), lambda i: (0, 0)),
            pl.BlockSpec((H, 2), lambda i: (0, 0)),
        ],
        out_specs=[
            pl.BlockSpec((rows_per_block, H), lambda i: (i, 0)),
            pl.BlockSpec((rows_per_block, 2), lambda i: (i, 0)),
        ],
        out_shape=[
            jax.ShapeDtypeStruct((N, H), jnp.float32),
            jax.ShapeDtypeStruct((N, 2), jnp.float32),
        ],
    )(x, W, a2)
    return h, av[:, 0], av[:, 1]


def _edge_body(srcs1, dsts1, srcs2, dsts2, asv_all, adv_all,
               h1a, h1b, h2a, h2b,
               out_part, s_part,
               asv_t, adv_t, src2d, dst2d, ex2d, rows0, rows1, zrow, zs,
               acc_sh, s_sh, sem0, sem1, sems0, sems1, sem_ss):
    c = lax.axis_index("c")
    t = lax.axis_index("s")
    wid = c * NS + t
    base = t * ROWS_PER_TILE

    def _zr(i, carry):
        for k in range(HH // 16):
            zrow[i, pl.ds(k * 16, 16)] = jnp.zeros((16,), jnp.float32)
        return carry
    lax.fori_loop(0, 128, _zr, 0)

    def _zs(i, carry):
        zs[pl.ds(i * 16, 16)] = jnp.zeros((16,), jnp.float32)
        return carry
    lax.fori_loop(0, ROWS_PER_TILE // 16, _zs, 0)

    def _zero_acc_slice():
        for k in range(ROWS_PER_TILE // 128):
            pltpu.sync_copy(zrow, acc_sh.at[pl.ds(base + k * 128, 128), :])

    def scale_chunk(rows_ref, j):
        j16 = jnp.full((16,), j, jnp.int32)

        def body(rq, carry):
            r0 = rq * 4
            exvs = [plsc.load_gather(
                ex2d, [j16, jnp.full((16,), r0 + u, jnp.int32)])
                for u in range(4)]
            for u in range(4):
                for k in range(HH // 16):
                    sl = pl.ds(k * 16, 16)
                    rows_ref[r0 + u, sl] = rows_ref[r0 + u, sl] * exvs[u]
            return carry
        lax.fori_loop(0, 32, body, 0)

    for p in range(2):
        pltpu.sync_copy(asv_all.at[p], asv_t)
        pltpu.sync_copy(adv_all.at[p], adv_t)
        h_halves = (h1a, h1b) if p == 0 else (h2a, h2b)
        srcs = srcs1 if p == 0 else srcs2
        dsts = dsts1 if p == 0 else dsts2
        for hop in range(3):
            s_i = p * 3 + hop
            _zero_acc_slice()
            pltpu.sync_copy(zs, s_sh.at[pl.ds(base, ROWS_PER_TILE)])

            pltpu.sync_copy(srcs.at[hop, wid], src2d)
            pltpu.sync_copy(dsts.at[hop, wid], dst2d)
            plsc.subcore_barrier()

            def _ex(j, carry):
                for k in range(8):
                    sl = pl.ds(k * 16, 16)
                    s16 = src2d[j, sl]
                    d16 = dst2d[j, sl]
                    av = plsc.load_gather(asv_t, [s16])
                    dv = plsc.load_gather(adv_t, [d16])
                    tt = av + dv
                    e = jnp.where(tt >= 0, tt, 0.2 * tt)
                    ex2d[j, sl] = jnp.exp(e)
                return carry
            lax.fori_loop(0, NCHUNK, _ex, 0)

            for half in range(2):
                h_hbm = h_halves[half]
                pltpu.async_copy(h_hbm.at[src2d.at[0]], rows0, sem0)
                pltpu.async_copy(h_hbm.at[src2d.at[1]], rows1, sem1)

                def _rows(jj, carry):
                    j0 = jj * 2
                    j1 = j0 + 1
                    jn0 = jnp.minimum(j0 + 2, NCHUNK - 1)
                    jn1 = jnp.minimum(j1 + 2, NCHUNK - 1)
                    pltpu.make_async_copy(h_hbm.at[src2d.at[j0]], rows0,
                                          sem0).wait()
                    scale_chunk(rows0, j0)
                    pltpu.async_copy(rows0, acc_sh.at[dst2d.at[j0]], sems0,
                                     add=True)
                    if half == 0:
                        pltpu.async_copy(ex2d.at[j0], s_sh.at[dst2d.at[j0]],
                                         sem_ss, add=True)
                    pltpu.make_async_copy(h_hbm.at[src2d.at[j1]], rows1,
                                          sem1).wait()
                    scale_chunk(rows1, j1)
                    pltpu.async_copy(rows1, acc_sh.at[dst2d.at[j1]], sems1,
                                     add=True)
                    if half == 0:
                        pltpu.async_copy(ex2d.at[j1], s_sh.at[dst2d.at[j1]],
                                         sem_ss, add=True)
                    pltpu.make_async_copy(rows0, acc_sh.at[dst2d.at[j0]],
                                          sems0).wait()
                    pltpu.async_copy(h_hbm.at[src2d.at[jn0]], rows0, sem0)
                    pltpu.make_async_copy(rows1, acc_sh.at[dst2d.at[j1]],
                                          sems1).wait()
                    pltpu.async_copy(h_hbm.at[src2d.at[jn1]], rows1, sem1)
                    if half == 0:
                        pltpu.make_async_copy(ex2d.at[j0],
                                              s_sh.at[dst2d.at[j0]],
                                              sem_ss).wait()
                        pltpu.make_async_copy(ex2d.at[j1],
                                              s_sh.at[dst2d.at[j1]],
                                              sem_ss).wait()
                    return carry
                lax.fori_loop(0, NCHUNK // 2, _rows, 0)
                pltpu.make_async_copy(h_hbm.at[src2d.at[NCHUNK - 1]], rows0,
                                      sem0).wait()
                pltpu.make_async_copy(h_hbm.at[src2d.at[NCHUNK - 1]], rows1,
                                      sem1).wait()

                plsc.subcore_barrier()
                pltpu.sync_copy(
                    acc_sh.at[pl.ds(base, ROWS_PER_TILE), :],
                    out_part.at[c, s_i, half, pl.ds(base, ROWS_PER_TILE), :])
                if half == 0:
                    _zero_acc_slice()
                    plsc.subcore_barrier()
            pltpu.sync_copy(s_sh.at[pl.ds(base, ROWS_PER_TILE)],
                            s_part.at[c, s_i, pl.ds(base, ROWS_PER_TILE)])


@functools.cache
def _build_edge_kernel():
    return functools.partial(
        pl.kernel,
        out_type=[
            jax.ShapeDtypeStruct((NC, 6, 2, N_PAD, HH), jnp.float32),
            jax.ShapeDtypeStruct((NC, 6, N_PAD), jnp.float32),
        ],
        mesh=plsc.VectorSubcoreMesh(core_axis_name="c", subcore_axis_name="s",
                                    num_cores=NC, num_subcores=NS),
        compiler_params=pltpu.CompilerParams(needs_layout_passes=False,
                                             use_tc_tiling_on_sc=False),
        scratch_types=[
            pltpu.VMEM((N_PAD,), jnp.float32),
            pltpu.VMEM((N_PAD,), jnp.float32),
            pltpu.VMEM((NCHUNK, 128), jnp.int32),
            pltpu.VMEM((NCHUNK, 128), jnp.int32),
            pltpu.VMEM((NCHUNK, 128), jnp.float32),
            pltpu.VMEM((128, HH), jnp.float32),
            pltpu.VMEM((128, HH), jnp.float32),
            pltpu.VMEM((128, HH), jnp.float32),
            pltpu.VMEM((ROWS_PER_TILE,), jnp.float32),
            pltpu.VMEM_SHARED((N_PAD, HH), jnp.float32),
            pltpu.VMEM_SHARED((N_PAD,), jnp.float32),
            pltpu.SemaphoreType.DMA,
            pltpu.SemaphoreType.DMA,
            pltpu.SemaphoreType.DMA,
            pltpu.SemaphoreType.DMA,
            pltpu.SemaphoreType.DMA,
        ],
    )(_edge_body)


def _pad_edges(ei):
    pad = EPAD - E
    ar = jnp.arange(pad, dtype=jnp.int32)
    src = jnp.concatenate([ei[0], (ar * 97) % N])
    dst = jnp.concatenate([ei[1], N + (ar % (N_PAD - N))])
    return (src.reshape(NW, NCHUNK, 128), dst.reshape(NW, NCHUNK, 128))


_RB = 1024


def _combine_body(op_ref, sp_ref, b1_ref, b2_ref, p1_ref, p2_ref,
                  pooled1_ref, pooled2_ref, cnt1_ref, cnt2_ref):
    i = pl.program_id(0)

    @pl.when(i == 0)
    def _():
        pooled1_ref[...] = jnp.zeros_like(pooled1_ref)
        pooled2_ref[...] = jnp.zeros_like(pooled2_ref)
        cnt1_ref[...] = jnp.zeros_like(cnt1_ref)
        cnt2_ref[...] = jnp.zeros_like(cnt2_ref)

    sp = sp_ref[...]
    ones = jnp.ones((_RB, H), jnp.float32)
    dn = (((0,), (0,)), ((), ()))
    for p in range(2):
        b_ref = b1_ref if p == 0 else b2_ref
        p_blk = (p1_ref if p == 0 else p2_ref)[...]
        cnt = lax.dot_general(p_blk, ones, dn,
                              preferred_element_type=jnp.float32)
        pooled_halves = []
        for half in range(2):
            xacc = jnp.zeros((_RB, HH), jnp.float32)
            for hop in range(3):
                s_i = p * 3 + hop
                raw = op_ref[0, s_i, half] + op_ref[1, s_i, half]
                s = sp[:, s_i:s_i + 1] + sp[:, 6 + s_i:7 + s_i]
                xh = raw / (s + 1e-16) + b_ref[:, half * HH:(half + 1) * HH]
                xacc = xacc + _leaky(xh, 0.01)
            pooled_halves.append(
                lax.dot_general(p_blk, xacc, dn,
                                preferred_element_type=jnp.float32))
        pooled_ref = pooled1_ref if p == 0 else pooled2_ref
        cnt_ref = cnt1_ref if p == 0 else cnt2_ref
        pooled_ref[:, 0:HH] += pooled_halves[0]
        pooled_ref[:, HH:H] += pooled_halves[1]
        cnt_ref[...] += cnt


def _combine(out_part, s_part, b1, b2, P1, P2):
    sT = s_part.reshape(NC * 6, N_PAD).T
    return pl.pallas_call(
        _combine_body,
        grid=(N_PAD // _RB,),
        in_specs=[
            pl.BlockSpec((NC, 6, 2, _RB, HH), lambda i: (0, 0, 0, i, 0)),
            pl.BlockSpec((_RB, NC * 6), lambda i: (i, 0)),
            pl.BlockSpec((1, H), lambda i: (0, 0)),
            pl.BlockSpec((1, H), lambda i: (0, 0)),
            pl.BlockSpec((_RB, G), lambda i: (i, 0)),
            pl.BlockSpec((_RB, G), lambda i: (i, 0)),
        ],
        out_specs=[
            pl.BlockSpec((G, H), lambda i: (0, 0)),
            pl.BlockSpec((G, H), lambda i: (0, 0)),
            pl.BlockSpec((G, H), lambda i: (0, 0)),
            pl.BlockSpec((G, H), lambda i: (0, 0)),
        ],
        out_shape=[jax.ShapeDtypeStruct((G, H), jnp.float32)] * 4,
    )(out_part, sT, b1.reshape(1, H), b2.reshape(1, H), P1, P2)


def _head_body(pooled1_ref, pooled2_ref, cnt1_ref, cnt2_ref,
               fc1p_w_ref, fc1p_b_ref, fc2p_w_ref, fc2p_b_ref,
               fcc1_w_ref, fcc1_b_ref, fcc2_w_ref, fcc2_b_ref,
               out_w_ref, out_b_ref, out_ref):
    x1 = pooled1_ref[...] / jnp.maximum(cnt1_ref[...], 1.0)
    x2 = pooled2_ref[...] / jnp.maximum(cnt2_ref[...], 1.0)
    z1 = _leaky(jnp.dot(x1, fc1p_w_ref[...],
                        preferred_element_type=jnp.float32)
                + fc1p_b_ref[...], 0.01)
    z2 = _leaky(jnp.dot(x2, fc2p_w_ref[...],
                        preferred_element_type=jnp.float32)
                + fc2p_b_ref[...], 0.01)
    xc = (jnp.dot(z1, fcc1_w_ref[0:H, :], preferred_element_type=jnp.float32)
          + jnp.dot(z2, fcc1_w_ref[H:2 * H, :],
                    preferred_element_type=jnp.float32)
          + fcc1_b_ref[...])
    xc = _leaky(xc, 0.01)
    xc = _leaky(jnp.dot(xc, fcc2_w_ref[...],
                        preferred_element_type=jnp.float32)
                + fcc2_b_ref[...], 0.01)
    z = jnp.dot(xc, out_w_ref[...],
                preferred_element_type=jnp.float32) + out_b_ref[...]
    out_ref[...] = 1.0 / (1.0 + jnp.exp(-z))


def _head(pooled1, pooled2, cnt1, cnt2, fc1p_w, fc1p_b, fc2p_w, fc2p_b,
          fcc1_w, fcc1_b, fcc2_w, fcc2_b, out_w, out_b):
    return pl.pallas_call(
        _head_body,
        out_shape=jax.ShapeDtypeStruct((G, 1), jnp.float32),
    )(pooled1, pooled2, cnt1, cnt2,
      fc1p_w, fc1p_b.reshape(1, 128), fc2p_w, fc2p_b.reshape(1, 128),
      fcc1_w, fcc1_b.reshape(1, 256), fcc2_w, fcc2_b.reshape(1, 64),
      out_w, out_b.reshape(1, 1))


def kernel(pro1_x, pro1_edge_index, pro1_two_hop_edge_index, pro1_three_hop_edge_index, pro1_batch, pro2_x, pro2_edge_index, pro2_two_hop_edge_index, pro2_three_hop_edge_index, pro2_batch, W1, a_src1, a_dst1, b1, W2, a_src2, a_dst2, b2, fc1p_w, fc1p_b, fc2p_w, fc2p_b, fcc1_w, fcc1_b, fcc2_w, fcc2_b, out_w, out_b):
    h1, asv1, adv1 = _projections(pro1_x, W1, a_src1, a_dst1)
    h2, asv2, adv2 = _projections(pro2_x, W2, a_src2, a_dst2)

    e1 = [_pad_edges(e) for e in
          (pro1_edge_index, pro1_two_hop_edge_index, pro1_three_hop_edge_index)]
    e2 = [_pad_edges(e) for e in
          (pro2_edge_index, pro2_two_hop_edge_index, pro2_three_hop_edge_index)]
    srcs1 = jnp.stack([s for s, _ in e1])
    dsts1 = jnp.stack([d for _, d in e1])
    srcs2 = jnp.stack([s for s, _ in e2])
    dsts2 = jnp.stack([d for _, d in e2])
    zpad = jnp.zeros((N_PAD - N,), jnp.float32)
    asv_all = jnp.stack([jnp.concatenate([asv1, zpad]),
                         jnp.concatenate([asv2, zpad])])
    adv_all = jnp.stack([jnp.concatenate([adv1, zpad]),
                         jnp.concatenate([adv2, zpad])])

    out_part, s_part = _build_edge_kernel()(
        srcs1, dsts1, srcs2, dsts2, asv_all, adv_all,
        h1[:, :HH], h1[:, HH:], h2[:, :HH], h2[:, HH:])

    ar = jnp.arange(G, dtype=jnp.int32)
    b1p = jnp.concatenate([pro1_batch, jnp.full((N_PAD - N,), G,
                                                jnp.int32)])
    b2p = jnp.concatenate([pro2_batch, jnp.full((N_PAD - N,), G,
                                                jnp.int32)])
    P1 = (b1p[:, None] == ar[None, :]).astype(jnp.float32)
    P2 = (b2p[:, None] == ar[None, :]).astype(jnp.float32)

    pooled1, pooled2, cnt1, cnt2 = _combine(out_part, s_part, b1, b2, P1, P2)
    return _head(pooled1, pooled2, cnt1, cnt2,
                 fc1p_w, fc1p_b, fc2p_w, fc2p_b,
                 fcc1_w, fcc1_b, fcc2_w, fcc2_b, out_w, out_b)


# speedup vs baseline: 1.1719x; 1.1719x over previous
import functools

import jax
import jax.numpy as jnp
from jax import lax
from jax.experimental import pallas as pl
from jax.experimental.pallas import tpu as pltpu
from jax.experimental.pallas import tpu_sc as plsc

N = 10000
D = 128
H = 128
G = 64
E = 320000

NC, NS, L = 2, 16, 16
NW = NC * NS
S_TILE = 10240
EPAD = NW * S_TILE
NCHUNK = S_TILE // 128
N_PAD = 10240
ROWS_PER_TILE = N_PAD // NS
HH = H // 2


def _leaky(x, slope):
    return jnp.where(x >= 0, x, slope * x)


def _proj_body(x_ref, w_ref, a_ref, h_ref, av_ref):
    h = jnp.dot(x_ref[...], w_ref[...], preferred_element_type=jnp.float32)
    h_ref[...] = h
    av_ref[...] = jnp.dot(h, a_ref[...], preferred_element_type=jnp.float32)


def _projections(x, W, a_src, a_dst, rows_per_block=2000):
    nb = N // rows_per_block
    a2 = jnp.stack([a_src, a_dst], axis=1)
    h, av = pl.pallas_call(
        _proj_body,
        grid=(nb,),
        in_specs=[
            pl.BlockSpec((rows_per_block, D), lambda i: (i, 0)),
            pl.BlockSpec((D, H), lambda i: (0, 0)),
            pl.BlockSpec((H, 2), lambda i: (0, 0)),
        ],
        out_specs=[
            pl.BlockSpec((rows_per_block, H), lambda i: (i, 0)),
            pl.BlockSpec((rows_per_block, 2), lambda i: (i, 0)),
        ],
        out_shape=[
            jax.ShapeDtypeStruct((N, H), jnp.float32),
            jax.ShapeDtypeStruct((N, 2), jnp.float32),
        ],
    )(x, W, a2)
    return h, av[:, 0], av[:, 1]


def _edge_body(srcs1, dsts1, srcs2, dsts2, asv_all, adv_all,
               h1a, h1b, h2a, h2b,
               out_part, s_part,
               asv_t, adv_t, src2d, dst2d, ex2d,
               rows0, rows1, rows2, zrow, zs,
               acc_sh, s_sh, gsem_arr, ssem_arr, sem_ss):
    c = lax.axis_index("c")
    t = lax.axis_index("s")
    wid = c * NS + t
    base = t * ROWS_PER_TILE

    def _zr(i, carry):
        for k in range(HH // 16):
            zrow[i, pl.ds(k * 16, 16)] = jnp.zeros((16,), jnp.float32)
        return carry
    lax.fori_loop(0, 128, _zr, 0)

    def _zs(i, carry):
        zs[pl.ds(i * 16, 16)] = jnp.zeros((16,), jnp.float32)
        return carry
    lax.fori_loop(0, ROWS_PER_TILE // 16, _zs, 0)

    def _zero_acc_slice():
        for k in range(ROWS_PER_TILE // 128):
            pltpu.sync_copy(zrow, acc_sh.at[pl.ds(base + k * 128, 128), :])

    def scale_chunk(rows_ref, j):
        j16 = jnp.full((16,), j, jnp.int32)

        def body(rq, carry):
            r0 = rq * 4
            exvs = [plsc.load_gather(
                ex2d, [j16, jnp.full((16,), r0 + u, jnp.int32)])
                for u in range(4)]
            for u in range(4):
                for k in range(HH // 16):
                    sl = pl.ds(k * 16, 16)
                    rows_ref[r0 + u, sl] = rows_ref[r0 + u, sl] * exvs[u]
            return carry
        lax.fori_loop(0, 32, body, 0)

    for p in range(2):
        pltpu.sync_copy(asv_all.at[p], asv_t)
        pltpu.sync_copy(adv_all.at[p], adv_t)
        h_halves = (h1a, h1b) if p == 0 else (h2a, h2b)
        srcs = srcs1 if p == 0 else srcs2
        dsts = dsts1 if p == 0 else dsts2
        def hop_body(hop, hcarry):
            s_i = p * 3 + hop
            _zero_acc_slice()
            pltpu.sync_copy(zs, s_sh.at[pl.ds(base, ROWS_PER_TILE)])

            pltpu.sync_copy(srcs.at[hop, wid], src2d)
            pltpu.sync_copy(dsts.at[hop, wid], dst2d)
            plsc.subcore_barrier()

            def _ex(j, carry):
                for k in range(8):
                    sl = pl.ds(k * 16, 16)
                    s16 = src2d[j, sl]
                    d16 = dst2d[j, sl]
                    av = plsc.load_gather(asv_t, [s16])
                    dv = plsc.load_gather(adv_t, [d16])
                    tt = av + dv
                    e = jnp.where(tt >= 0, tt, 0.2 * tt)
                    ex2d[j, sl] = jnp.exp(e)
                return carry
            lax.fori_loop(0, NCHUNK, _ex, 0)

            for half in range(2):
                h_hbm = h_halves[half]
                bufs = (rows0, rows1, rows2)
                gsem = tuple(gsem_arr.at[u] for u in range(3))
                ssem = tuple(ssem_arr.at[u] for u in range(3))

                def fire_gather(m, u):
                    pltpu.async_copy(h_hbm.at[src2d.at[m]], bufs[u], gsem[u])

                def wait_gather(m, u):
                    pltpu.make_async_copy(h_hbm.at[src2d.at[m]], bufs[u],
                                          gsem[u]).wait()

                def fire_scatter(m, u):
                    pltpu.async_copy(bufs[u], acc_sh.at[dst2d.at[m]],
                                     ssem[u], add=True)
                    if half == 0:
                        pltpu.async_copy(ex2d.at[m], s_sh.at[dst2d.at[m]],
                                         sem_ss, add=True)

                def wait_scatter(m, u):
                    pltpu.make_async_copy(bufs[u], acc_sh.at[dst2d.at[m]],
                                          ssem[u]).wait()
                    if half == 0:
                        pltpu.make_async_copy(ex2d.at[m],
                                              s_sh.at[dst2d.at[m]],
                                              sem_ss).wait()

                def do_slot(j, u, wait_prev=True, fire_next=True):
                    wait_gather(j, u)
                    scale_chunk(bufs[u], j)
                    fire_scatter(j, u)
                    if wait_prev:
                        wait_scatter(j - 1, (u + 2) % 3)
                    if fire_next:
                        fire_gather(j + 2, (u + 2) % 3)

                fire_gather(0, 0)
                fire_gather(1, 1)
                do_slot(0, 0, wait_prev=False)
                do_slot(1, 1)

                def _mid(jj, carry):
                    j = 2 + jj * 3
                    do_slot(j, 2)
                    do_slot(j + 1, 0)
                    do_slot(j + 2, 1)
                    return carry
                lax.fori_loop(0, (NCHUNK - 5) // 3, _mid, 0)

                do_slot(NCHUNK - 3, 2)
                do_slot(NCHUNK - 2, 0, fire_next=False)
                do_slot(NCHUNK - 1, 1, fire_next=False)
                wait_scatter(NCHUNK - 1, 1)
                plsc.subcore_barrier()
                pltpu.sync_copy(
                    acc_sh.at[pl.ds(base, ROWS_PER_TILE), :],
                    out_part.at[c, s_i, half, pl.ds(base, ROWS_PER_TILE), :])
                if half == 0:
                    _zero_acc_slice()
                    plsc.subcore_barrier()
            pltpu.sync_copy(s_sh.at[pl.ds(base, ROWS_PER_TILE)],
                            s_part.at[c, s_i, pl.ds(base, ROWS_PER_TILE)])
            return hcarry
        lax.fori_loop(0, 3, hop_body, 0)


@functools.cache
def _build_edge_kernel():
    return functools.partial(
        pl.kernel,
        out_type=[
            jax.ShapeDtypeStruct((NC, 6, 2, N_PAD, HH), jnp.float32),
            jax.ShapeDtypeStruct((NC, 6, N_PAD), jnp.float32),
        ],
        mesh=plsc.VectorSubcoreMesh(core_axis_name="c", subcore_axis_name="s",
                                    num_cores=NC, num_subcores=NS),
        compiler_params=pltpu.CompilerParams(needs_layout_passes=False,
                                             use_tc_tiling_on_sc=False),
        scratch_types=[
            pltpu.VMEM((N_PAD,), jnp.float32),
            pltpu.VMEM((N_PAD,), jnp.float32),
            pltpu.VMEM((NCHUNK, 128), jnp.int32),
            pltpu.VMEM((NCHUNK, 128), jnp.int32),
            pltpu.VMEM((NCHUNK, 128), jnp.float32),
            pltpu.VMEM((128, HH), jnp.float32),
            pltpu.VMEM((128, HH), jnp.float32),
            pltpu.VMEM((128, HH), jnp.float32),
            pltpu.VMEM((128, HH), jnp.float32),
            pltpu.VMEM((ROWS_PER_TILE,), jnp.float32),
            pltpu.VMEM_SHARED((N_PAD, HH), jnp.float32),
            pltpu.VMEM_SHARED((N_PAD,), jnp.float32),
            pltpu.SemaphoreType.DMA((3,)),
            pltpu.SemaphoreType.DMA((3,)),
            pltpu.SemaphoreType.DMA,
        ],
    )(_edge_body)


def _pad_edges(ei):
    pad = EPAD - E
    ar = jnp.arange(pad, dtype=jnp.int32)
    src = jnp.concatenate([ei[0], (ar * 97) % N])
    dst = jnp.concatenate([ei[1], N + (ar % (N_PAD - N))])
    return (src.reshape(NW, NCHUNK, 128), dst.reshape(NW, NCHUNK, 128))


_RB = 1024


def _combine_body(op_ref, sp_ref, b1_ref, b2_ref, p1_ref, p2_ref,
                  pooled1_ref, pooled2_ref, cnt1_ref, cnt2_ref):
    i = pl.program_id(0)

    @pl.when(i == 0)
    def _():
        pooled1_ref[...] = jnp.zeros_like(pooled1_ref)
        pooled2_ref[...] = jnp.zeros_like(pooled2_ref)
        cnt1_ref[...] = jnp.zeros_like(cnt1_ref)
        cnt2_ref[...] = jnp.zeros_like(cnt2_ref)

    sp = sp_ref[...]
    ones = jnp.ones((_RB, H), jnp.float32)
    dn = (((0,), (0,)), ((), ()))
    for p in range(2):
        b_ref = b1_ref if p == 0 else b2_ref
        p_blk = (p1_ref if p == 0 else p2_ref)[...]
        cnt = lax.dot_general(p_blk, ones, dn,
                              preferred_element_type=jnp.float32)
        pooled_halves = []
        for half in range(2):
            xacc = jnp.zeros((_RB, HH), jnp.float32)
            for hop in range(3):
                s_i = p * 3 + hop
                raw = op_ref[0, s_i, half] + op_ref[1, s_i, half]
                s = sp[:, s_i:s_i + 1] + sp[:, 6 + s_i:7 + s_i]
                xh = raw / (s + 1e-16) + b_ref[:, half * HH:(half + 1) * HH]
                xacc = xacc + _leaky(xh, 0.01)
            pooled_halves.append(
                lax.dot_general(p_blk, xacc, dn,
                                preferred_element_type=jnp.float32))
        pooled_ref = pooled1_ref if p == 0 else pooled2_ref
        cnt_ref = cnt1_ref if p == 0 else cnt2_ref
        pooled_ref[:, 0:HH] += pooled_halves[0]
        pooled_ref[:, HH:H] += pooled_halves[1]
        cnt_ref[...] += cnt


def _combine(out_part, s_part, b1, b2, P1, P2):
    sT = s_part.reshape(NC * 6, N_PAD).T
    return pl.pallas_call(
        _combine_body,
        grid=(N_PAD // _RB,),
        in_specs=[
            pl.BlockSpec((NC, 6, 2, _RB, HH), lambda i: (0, 0, 0, i, 0)),
            pl.BlockSpec((_RB, NC * 6), lambda i: (i, 0)),
            pl.BlockSpec((1, H), lambda i: (0, 0)),
            pl.BlockSpec((1, H), lambda i: (0, 0)),
            pl.BlockSpec((_RB, G), lambda i: (i, 0)),
            pl.BlockSpec((_RB, G), lambda i: (i, 0)),
        ],
        out_specs=[
            pl.BlockSpec((G, H), lambda i: (0, 0)),
            pl.BlockSpec((G, H), lambda i: (0, 0)),
            pl.BlockSpec((G, H), lambda i: (0, 0)),
            pl.BlockSpec((G, H), lambda i: (0, 0)),
        ],
        out_shape=[jax.ShapeDtypeStruct((G, H), jnp.float32)] * 4,
    )(out_part, sT, b1.reshape(1, H), b2.reshape(1, H), P1, P2)


def _head_body(pooled1_ref, pooled2_ref, cnt1_ref, cnt2_ref,
               fc1p_w_ref, fc1p_b_ref, fc2p_w_ref, fc2p_b_ref,
               fcc1_w_ref, fcc1_b_ref, fcc2_w_ref, fcc2_b_ref,
               out_w_ref, out_b_ref, out_ref):
    x1 = pooled1_ref[...] / jnp.maximum(cnt1_ref[...], 1.0)
    x2 = pooled2_ref[...] / jnp.maximum(cnt2_ref[...], 1.0)
    z1 = _leaky(jnp.dot(x1, fc1p_w_ref[...],
                        preferred_element_type=jnp.float32)
                + fc1p_b_ref[...], 0.01)
    z2 = _leaky(jnp.dot(x2, fc2p_w_ref[...],
                        preferred_element_type=jnp.float32)
                + fc2p_b_ref[...], 0.01)
    xc = (jnp.dot(z1, fcc1_w_ref[0:H, :], preferred_element_type=jnp.float32)
          + jnp.dot(z2, fcc1_w_ref[H:2 * H, :],
                    preferred_element_type=jnp.float32)
          + fcc1_b_ref[...])
    xc = _leaky(xc, 0.01)
    xc = _leaky(jnp.dot(xc, fcc2_w_ref[...],
                        preferred_element_type=jnp.float32)
                + fcc2_b_ref[...], 0.01)
    z = jnp.dot(xc, out_w_ref[...],
                preferred_element_type=jnp.float32) + out_b_ref[...]
    out_ref[...] = 1.0 / (1.0 + jnp.exp(-z))


def _head(pooled1, pooled2, cnt1, cnt2, fc1p_w, fc1p_b, fc2p_w, fc2p_b,
          fcc1_w, fcc1_b, fcc2_w, fcc2_b, out_w, out_b):
    return pl.pallas_call(
        _head_body,
        out_shape=jax.ShapeDtypeStruct((G, 1), jnp.float32),
    )(pooled1, pooled2, cnt1, cnt2,
      fc1p_w, fc1p_b.reshape(1, 128), fc2p_w, fc2p_b.reshape(1, 128),
      fcc1_w, fcc1_b.reshape(1, 256), fcc2_w, fcc2_b.reshape(1, 64),
      out_w, out_b.reshape(1, 1))


def kernel(pro1_x, pro1_edge_index, pro1_two_hop_edge_index, pro1_three_hop_edge_index, pro1_batch, pro2_x, pro2_edge_index, pro2_two_hop_edge_index, pro2_three_hop_edge_index, pro2_batch, W1, a_src1, a_dst1, b1, W2, a_src2, a_dst2, b2, fc1p_w, fc1p_b, fc2p_w, fc2p_b, fcc1_w, fcc1_b, fcc2_w, fcc2_b, out_w, out_b):
    h1, asv1, adv1 = _projections(pro1_x, W1, a_src1, a_dst1)
    h2, asv2, adv2 = _projections(pro2_x, W2, a_src2, a_dst2)

    e1 = [_pad_edges(e) for e in
          (pro1_edge_index, pro1_two_hop_edge_index, pro1_three_hop_edge_index)]
    e2 = [_pad_edges(e) for e in
          (pro2_edge_index, pro2_two_hop_edge_index, pro2_three_hop_edge_index)]
    srcs1 = jnp.stack([s for s, _ in e1])
    dsts1 = jnp.stack([d for _, d in e1])
    srcs2 = jnp.stack([s for s, _ in e2])
    dsts2 = jnp.stack([d for _, d in e2])
    zpad = jnp.zeros((N_PAD - N,), jnp.float32)
    asv_all = jnp.stack([jnp.concatenate([asv1, zpad]),
                         jnp.concatenate([asv2, zpad])])
    adv_all = jnp.stack([jnp.concatenate([adv1, zpad]),
                         jnp.concatenate([adv2, zpad])])

    out_part, s_part = _build_edge_kernel()(
        srcs1, dsts1, srcs2, dsts2, asv_all, adv_all,
        h1[:, :HH], h1[:, HH:], h2[:, :HH], h2[:, HH:])

    ar = jnp.arange(G, dtype=jnp.int32)
    b1p = jnp.concatenate([pro1_batch, jnp.full((N_PAD - N,), G,
                                                jnp.int32)])
    b2p = jnp.concatenate([pro2_batch, jnp.full((N_PAD - N,), G,
                                                jnp.int32)])
    P1 = (b1p[:, None] == ar[None, :]).astype(jnp.float32)
    P2 = (b2p[:, None] == ar[None, :]).astype(jnp.float32)

    pooled1, pooled2, cnt1, cnt2 = _combine(out_part, s_part, b1, b2, P1, P2)
    return _head(pooled1, pooled2, cnt1, cnt2,
                 fc1p_w, fc1p_b, fc2p_w, fc2p_b,
                 fcc1_w, fcc1_b, fcc2_w, fcc2_b, out_w, out_b)


# speedup vs baseline: 1.1965x; 1.0211x over previous
import functools

import jax
import jax.numpy as jnp
from jax import lax
from jax.experimental import pallas as pl
from jax.experimental.pallas import tpu as pltpu
from jax.experimental.pallas import tpu_sc as plsc

N = 10000
D = 128
H = 128
G = 64
E = 320000

NC, NS, L = 2, 16, 16
NW = NC * NS
S_TILE = 10240
EPAD = NW * S_TILE
NCHUNK = S_TILE // 128
N_PAD = 10240
ROWS_PER_TILE = N_PAD // NS
HH = H // 2


def _leaky(x, slope):
    return jnp.where(x >= 0, x, slope * x)


def _proj_body(x_ref, w_ref, a_ref, h_ref, av_ref):
    h = jnp.dot(x_ref[...], w_ref[...], preferred_element_type=jnp.float32)
    h_ref[...] = h
    av_ref[...] = jnp.dot(h, a_ref[...], preferred_element_type=jnp.float32)


def _projections(x, W, a_src, a_dst, rows_per_block=2000):
    nb = N // rows_per_block
    a2 = jnp.stack([a_src, a_dst], axis=1)
    h, av = pl.pallas_call(
        _proj_body,
        grid=(nb,),
        in_specs=[
            pl.BlockSpec((rows_per_block, D), lambda i: (i, 0)),
            pl.BlockSpec((D, H), lambda i: (0, 0)),
            pl.BlockSpec((H, 2), lambda i: (0, 0)),
        ],
        out_specs=[
            pl.BlockSpec((rows_per_block, H), lambda i: (i, 0)),
            pl.BlockSpec((rows_per_block, 2), lambda i: (i, 0)),
        ],
        out_shape=[
            jax.ShapeDtypeStruct((N, H), jnp.float32),
            jax.ShapeDtypeStruct((N, 2), jnp.float32),
        ],
    )(x, W, a2)
    return h, av[:, 0], av[:, 1]


def _edge_body(srcs1, dsts1, srcs2, dsts2, asv_all, adv_all,
               h1a, h1b, h2a, h2b,
               out_part, s_part,
               asv_t, adv_t, src2d, dst2d, ex2d,
               rows0, rows1, rows2, zrow, zs,
               acc_sh, s_sh, gsem_arr, ssem_arr, sem_ss):
    c = lax.axis_index("c")
    t = lax.axis_index("s")
    wid = c * NS + t
    base = t * ROWS_PER_TILE

    def _zr(i, carry):
        for k in range(HH // 16):
            zrow[i, pl.ds(k * 16, 16)] = jnp.zeros((16,), jnp.float32)
        return carry
    lax.fori_loop(0, 128, _zr, 0)

    def _zs(i, carry):
        zs[pl.ds(i * 16, 16)] = jnp.zeros((16,), jnp.float32)
        return carry
    lax.fori_loop(0, ROWS_PER_TILE // 16, _zs, 0)

    def _zero_acc_slice():
        for k in range(ROWS_PER_TILE // 128):
            pltpu.sync_copy(zrow, acc_sh.at[pl.ds(base + k * 128, 128), :])

    def scale_chunk(rows_ref, j):
        j16 = jnp.full((16,), j, jnp.int32)

        def body(rq, carry):
            r0 = rq * 8
            exvs = [plsc.load_gather(
                ex2d, [j16, jnp.full((16,), r0 + u, jnp.int32)])
                for u in range(8)]
            for u in range(8):
                for k in range(HH // 16):
                    sl = pl.ds(k * 16, 16)
                    rows_ref[r0 + u, sl] = rows_ref[r0 + u, sl] * exvs[u]
            return carry
        lax.fori_loop(0, 16, body, 0)

    for p in range(2):
        pltpu.sync_copy(asv_all.at[p], asv_t)
        pltpu.sync_copy(adv_all.at[p], adv_t)
        h_halves = (h1a, h1b) if p == 0 else (h2a, h2b)
        srcs = srcs1 if p == 0 else srcs2
        dsts = dsts1 if p == 0 else dsts2
        def hop_body(hop, hcarry):
            s_i = p * 3 + hop
            _zero_acc_slice()
            pltpu.sync_copy(zs, s_sh.at[pl.ds(base, ROWS_PER_TILE)])

            pltpu.sync_copy(srcs.at[hop, wid], src2d)
            pltpu.sync_copy(dsts.at[hop, wid], dst2d)
            plsc.subcore_barrier()

            def _ex(j, carry):
                for k in range(8):
                    sl = pl.ds(k * 16, 16)
                    s16 = src2d[j, sl]
                    d16 = dst2d[j, sl]
                    av = plsc.load_gather(asv_t, [s16])
                    dv = plsc.load_gather(adv_t, [d16])
                    tt = av + dv
                    e = jnp.where(tt >= 0, tt, 0.2 * tt)
                    ex2d[j, sl] = jnp.exp(e)
                return carry
            lax.fori_loop(0, NCHUNK, _ex, 0)

            for half in range(2):
                h_hbm = h_halves[half]
                bufs = (rows0, rows1, rows2)
                gsem = tuple(gsem_arr.at[u] for u in range(3))
                ssem = tuple(ssem_arr.at[u] for u in range(3))

                def fire_gather(m, u):
                    pltpu.async_copy(h_hbm.at[src2d.at[m]], bufs[u], gsem[u])

                def wait_gather(m, u):
                    pltpu.make_async_copy(h_hbm.at[src2d.at[m]], bufs[u],
                                          gsem[u]).wait()

                def fire_scatter(m, u):
                    pltpu.async_copy(bufs[u], acc_sh.at[dst2d.at[m]],
                                     ssem[u], add=True)
                    if half == 0:
                        pltpu.async_copy(ex2d.at[m], s_sh.at[dst2d.at[m]],
                                         sem_ss, add=True)

                def wait_scatter(m, u):
                    pltpu.make_async_copy(bufs[u], acc_sh.at[dst2d.at[m]],
                                          ssem[u]).wait()
                    if half == 0:
                        pltpu.make_async_copy(ex2d.at[m],
                                              s_sh.at[dst2d.at[m]],
                                              sem_ss).wait()

                def do_slot(j, u, wait_prev=True, fire_next=True):
                    wait_gather(j, u)
                    scale_chunk(bufs[u], j)
                    fire_scatter(j, u)
                    if wait_prev:
                        wait_scatter(j - 1, (u + 2) % 3)
                    if fire_next:
                        fire_gather(j + 2, (u + 2) % 3)

                fire_gather(0, 0)
                fire_gather(1, 1)
                do_slot(0, 0, wait_prev=False)
                do_slot(1, 1)

                def _mid(jj, carry):
                    j = 2 + jj * 3
                    do_slot(j, 2)
                    do_slot(j + 1, 0)
                    do_slot(j + 2, 1)
                    return carry
                lax.fori_loop(0, (NCHUNK - 5) // 3, _mid, 0)

                do_slot(NCHUNK - 3, 2)
                do_slot(NCHUNK - 2, 0, fire_next=False)
                do_slot(NCHUNK - 1, 1, fire_next=False)
                wait_scatter(NCHUNK - 1, 1)
                plsc.subcore_barrier()
                pltpu.sync_copy(
                    acc_sh.at[pl.ds(base, ROWS_PER_TILE), :],
                    out_part.at[c, s_i, half, pl.ds(base, ROWS_PER_TILE), :])
                if half == 0:
                    _zero_acc_slice()
                    plsc.subcore_barrier()
            pltpu.sync_copy(s_sh.at[pl.ds(base, ROWS_PER_TILE)],
                            s_part.at[c, s_i, pl.ds(base, ROWS_PER_TILE)])
            return hcarry
        lax.fori_loop(0, 3, hop_body, 0)


@functools.cache
def _build_edge_kernel():
    return functools.partial(
        pl.kernel,
        out_type=[
            jax.ShapeDtypeStruct((NC, 6, 2, N_PAD, HH), jnp.float32),
            jax.ShapeDtypeStruct((NC, 6, N_PAD), jnp.float32),
        ],
        mesh=plsc.VectorSubcoreMesh(core_axis_name="c", subcore_axis_name="s",
                                    num_cores=NC, num_subcores=NS),
        compiler_params=pltpu.CompilerParams(needs_layout_passes=False,
                                             use_tc_tiling_on_sc=False),
        scratch_types=[
            pltpu.VMEM((N_PAD,), jnp.float32),
            pltpu.VMEM((N_PAD,), jnp.float32),
            pltpu.VMEM((NCHUNK, 128), jnp.int32),
            pltpu.VMEM((NCHUNK, 128), jnp.int32),
            pltpu.VMEM((NCHUNK, 128), jnp.float32),
            pltpu.VMEM((128, HH), jnp.float32),
            pltpu.VMEM((128, HH), jnp.float32),
            pltpu.VMEM((128, HH), jnp.float32),
            pltpu.VMEM((128, HH), jnp.float32),
            pltpu.VMEM((ROWS_PER_TILE,), jnp.float32),
            pltpu.VMEM_SHARED((N_PAD, HH), jnp.float32),
            pltpu.VMEM_SHARED((N_PAD,), jnp.float32),
            pltpu.SemaphoreType.DMA((3,)),
            pltpu.SemaphoreType.DMA((3,)),
            pltpu.SemaphoreType.DMA,
        ],
    )(_edge_body)


def _pad_edges(ei):
    pad = EPAD - E
    ar = jnp.arange(pad, dtype=jnp.int32)
    src = jnp.concatenate([ei[0], (ar * 97) % N])
    dst = jnp.concatenate([ei[1], N + (ar % (N_PAD - N))])
    return (src.reshape(NW, NCHUNK, 128), dst.reshape(NW, NCHUNK, 128))


_RB = 1024


def _combine_body(op_ref, sp_ref, b1_ref, b2_ref, p1_ref, p2_ref,
                  pooled1_ref, pooled2_ref, cnt1_ref, cnt2_ref):
    i = pl.program_id(0)

    @pl.when(i == 0)
    def _():
        pooled1_ref[...] = jnp.zeros_like(pooled1_ref)
        pooled2_ref[...] = jnp.zeros_like(pooled2_ref)
        cnt1_ref[...] = jnp.zeros_like(cnt1_ref)
        cnt2_ref[...] = jnp.zeros_like(cnt2_ref)

    sp = sp_ref[...]
    ones = jnp.ones((_RB, H), jnp.float32)
    dn = (((0,), (0,)), ((), ()))
    for p in range(2):
        b_ref = b1_ref if p == 0 else b2_ref
        p_blk = (p1_ref if p == 0 else p2_ref)[...]
        cnt = lax.dot_general(p_blk, ones, dn,
                              preferred_element_type=jnp.float32)
        pooled_halves = []
        for half in range(2):
            xacc = jnp.zeros((_RB, HH), jnp.float32)
            for hop in range(3):
                s_i = p * 3 + hop
                raw = op_ref[0, s_i, half] + op_ref[1, s_i, half]
                s = sp[:, s_i:s_i + 1] + sp[:, 6 + s_i:7 + s_i]
                xh = raw / (s + 1e-16) + b_ref[:, half * HH:(half + 1) * HH]
                xacc = xacc + _leaky(xh, 0.01)
            pooled_halves.append(
                lax.dot_general(p_blk, xacc, dn,
                                preferred_element_type=jnp.float32))
        pooled_ref = pooled1_ref if p == 0 else pooled2_ref
        cnt_ref = cnt1_ref if p == 0 else cnt2_ref
        pooled_ref[:, 0:HH] += pooled_halves[0]
        pooled_ref[:, HH:H] += pooled_halves[1]
        cnt_ref[...] += cnt


def _combine(out_part, s_part, b1, b2, P1, P2):
    sT = s_part.reshape(NC * 6, N_PAD).T
    return pl.pallas_call(
        _combine_body,
        grid=(N_PAD // _RB,),
        in_specs=[
            pl.BlockSpec((NC, 6, 2, _RB, HH), lambda i: (0, 0, 0, i, 0)),
            pl.BlockSpec((_RB, NC * 6), lambda i: (i, 0)),
            pl.BlockSpec((1, H), lambda i: (0, 0)),
            pl.BlockSpec((1, H), lambda i: (0, 0)),
            pl.BlockSpec((_RB, G), lambda i: (i, 0)),
            pl.BlockSpec((_RB, G), lambda i: (i, 0)),
        ],
        out_specs=[
            pl.BlockSpec((G, H), lambda i: (0, 0)),
            pl.BlockSpec((G, H), lambda i: (0, 0)),
            pl.BlockSpec((G, H), lambda i: (0, 0)),
            pl.BlockSpec((G, H), lambda i: (0, 0)),
        ],
        out_shape=[jax.ShapeDtypeStruct((G, H), jnp.float32)] * 4,
    )(out_part, sT, b1.reshape(1, H), b2.reshape(1, H), P1, P2)


def _head_body(pooled1_ref, pooled2_ref, cnt1_ref, cnt2_ref,
               fc1p_w_ref, fc1p_b_ref, fc2p_w_ref, fc2p_b_ref,
               fcc1_w_ref, fcc1_b_ref, fcc2_w_ref, fcc2_b_ref,
               out_w_ref, out_b_ref, out_ref):
    x1 = pooled1_ref[...] / jnp.maximum(cnt1_ref[...], 1.0)
    x2 = pooled2_ref[...] / jnp.maximum(cnt2_ref[...], 1.0)
    z1 = _leaky(jnp.dot(x1, fc1p_w_ref[...],
                        preferred_element_type=jnp.float32)
                + fc1p_b_ref[...], 0.01)
    z2 = _leaky(jnp.dot(x2, fc2p_w_ref[...],
                        preferred_element_type=jnp.float32)
                + fc2p_b_ref[...], 0.01)
    xc = (jnp.dot(z1, fcc1_w_ref[0:H, :], preferred_element_type=jnp.float32)
          + jnp.dot(z2, fcc1_w_ref[H:2 * H, :],
                    preferred_element_type=jnp.float32)
          + fcc1_b_ref[...])
    xc = _leaky(xc, 0.01)
    xc = _leaky(jnp.dot(xc, fcc2_w_ref[...],
                        preferred_element_type=jnp.float32)
                + fcc2_b_ref[...], 0.01)
    z = jnp.dot(xc, out_w_ref[...],
                preferred_element_type=jnp.float32) + out_b_ref[...]
    out_ref[...] = 1.0 / (1.0 + jnp.exp(-z))


def _head(pooled1, pooled2, cnt1, cnt2, fc1p_w, fc1p_b, fc2p_w, fc2p_b,
          fcc1_w, fcc1_b, fcc2_w, fcc2_b, out_w, out_b):
    return pl.pallas_call(
        _head_body,
        out_shape=jax.ShapeDtypeStruct((G, 1), jnp.float32),
    )(pooled1, pooled2, cnt1, cnt2,
      fc1p_w, fc1p_b.reshape(1, 128), fc2p_w, fc2p_b.reshape(1, 128),
      fcc1_w, fcc1_b.reshape(1, 256), fcc2_w, fcc2_b.reshape(1, 64),
      out_w, out_b.reshape(1, 1))


def kernel(pro1_x, pro1_edge_index, pro1_two_hop_edge_index, pro1_three_hop_edge_index, pro1_batch, pro2_x, pro2_edge_index, pro2_two_hop_edge_index, pro2_three_hop_edge_index, pro2_batch, W1, a_src1, a_dst1, b1, W2, a_src2, a_dst2, b2, fc1p_w, fc1p_b, fc2p_w, fc2p_b, fcc1_w, fcc1_b, fcc2_w, fcc2_b, out_w, out_b):
    h1, asv1, adv1 = _projections(pro1_x, W1, a_src1, a_dst1)
    h2, asv2, adv2 = _projections(pro2_x, W2, a_src2, a_dst2)

    e1 = [_pad_edges(e) for e in
          (pro1_edge_index, pro1_two_hop_edge_index, pro1_three_hop_edge_index)]
    e2 = [_pad_edges(e) for e in
          (pro2_edge_index, pro2_two_hop_edge_index, pro2_three_hop_edge_index)]
    srcs1 = jnp.stack([s for s, _ in e1])
    dsts1 = jnp.stack([d for _, d in e1])
    srcs2 = jnp.stack([s for s, _ in e2])
    dsts2 = jnp.stack([d for _, d in e2])
    zpad = jnp.zeros((N_PAD - N,), jnp.float32)
    asv_all = jnp.stack([jnp.concatenate([asv1, zpad]),
                         jnp.concatenate([asv2, zpad])])
    adv_all = jnp.stack([jnp.concatenate([adv1, zpad]),
                         jnp.concatenate([adv2, zpad])])

    out_part, s_part = _build_edge_kernel()(
        srcs1, dsts1, srcs2, dsts2, asv_all, adv_all,
        h1[:, :HH], h1[:, HH:], h2[:, :HH], h2[:, HH:])

    ar = jnp.arange(G, dtype=jnp.int32)
    b1p = jnp.concatenate([pro1_batch, jnp.full((N_PAD - N,), G,
                                                jnp.int32)])
    b2p = jnp.concatenate([pro2_batch, jnp.full((N_PAD - N,), G,
                                                jnp.int32)])
    P1 = (b1p[:, None] == ar[None, :]).astype(jnp.float32)
    P2 = (b2p[:, None] == ar[None, :]).astype(jnp.float32)

    pooled1, pooled2, cnt1, cnt2 = _combine(out_part, s_part, b1, b2, P1, P2)
    return _head(pooled1, pooled2, cnt1, cnt2,
                 fc1p_w, fc1p_b, fc2p_w, fc2p_b,
                 fcc1_w, fcc1_b, fcc2_w, fcc2_b, out_w, out_b)


# speedup vs baseline: 1.2101x; 1.0113x over previous
import functools

import jax
import jax.numpy as jnp
from jax import lax
from jax.experimental import pallas as pl
from jax.experimental.pallas import tpu as pltpu
from jax.experimental.pallas import tpu_sc as plsc

N = 10000
D = 128
H = 128
G = 64
E = 320000

NC, NS, L = 2, 16, 16
NW = NC * NS
S_TILE = 10240
EPAD = NW * S_TILE
NCHUNK = S_TILE // 128
N_PAD = 10240
ROWS_PER_TILE = N_PAD // NS
HH = H // 2


def _leaky(x, slope):
    return jnp.where(x >= 0, x, slope * x)


def _proj_body(x_ref, w_ref, a_ref, h_ref, av_ref):
    h = jnp.dot(x_ref[...], w_ref[...], preferred_element_type=jnp.float32)
    h_ref[...] = h
    av_ref[...] = jnp.dot(h, a_ref[...], preferred_element_type=jnp.float32)


def _projections(x, W, a_src, a_dst, rows_per_block=2000):
    nb = N // rows_per_block
    a2 = jnp.stack([a_src, a_dst], axis=1)
    h, av = pl.pallas_call(
        _proj_body,
        grid=(nb,),
        in_specs=[
            pl.BlockSpec((rows_per_block, D), lambda i: (i, 0)),
            pl.BlockSpec((D, H), lambda i: (0, 0)),
            pl.BlockSpec((H, 2), lambda i: (0, 0)),
        ],
        out_specs=[
            pl.BlockSpec((rows_per_block, H), lambda i: (i, 0)),
            pl.BlockSpec((rows_per_block, 2), lambda i: (i, 0)),
        ],
        out_shape=[
            jax.ShapeDtypeStruct((N, H), jnp.float32),
            jax.ShapeDtypeStruct((N, 2), jnp.float32),
        ],
    )(x, W, a2)
    return h, av[:, 0], av[:, 1]


def _edge_body(srcs1, dsts1, srcs2, dsts2, asv_all, adv_all,
               h1a, h1b, h2a, h2b,
               out_part, s_part,
               asv_t, adv_t, src2d, dst2d, ex2d,
               rows0, rows1, rows2, zrow, zs,
               acc_sh, s_sh, gsem_arr, ssem_arr, sem_ss, sem_out):
    c = lax.axis_index("c")
    t = lax.axis_index("s")
    wid = c * NS + t
    base = t * ROWS_PER_TILE

    def _zr(i, carry):
        for k in range(HH // 16):
            zrow[i, pl.ds(k * 16, 16)] = jnp.zeros((16,), jnp.float32)
        return carry
    lax.fori_loop(0, 128, _zr, 0)

    def _zs(i, carry):
        zs[pl.ds(i * 16, 16)] = jnp.zeros((16,), jnp.float32)
        return carry
    lax.fori_loop(0, ROWS_PER_TILE // 16, _zs, 0)

    def _wait_copyout():
        pltpu.make_async_copy(
            acc_sh.at[pl.ds(base, ROWS_PER_TILE), :],
            out_part.at[0, 0, 1, pl.ds(base, ROWS_PER_TILE), :],
            sem_out).wait()
        pltpu.make_async_copy(
            s_sh.at[pl.ds(base, ROWS_PER_TILE)],
            s_part.at[0, 0, pl.ds(base, ROWS_PER_TILE)],
            sem_out).wait()

    def _zero_acc_slice():
        for k in range(ROWS_PER_TILE // 128):
            pltpu.sync_copy(zrow, acc_sh.at[pl.ds(base + k * 128, 128), :])

    def scale_chunk(rows_ref, j):
        j16 = jnp.full((16,), j, jnp.int32)

        def body(rq, carry):
            r0 = rq * 8
            exvs = [plsc.load_gather(
                ex2d, [j16, jnp.full((16,), r0 + u, jnp.int32)])
                for u in range(8)]
            for u in range(8):
                for k in range(HH // 16):
                    sl = pl.ds(k * 16, 16)
                    rows_ref[r0 + u, sl] = rows_ref[r0 + u, sl] * exvs[u]
            return carry
        lax.fori_loop(0, 16, body, 0)

    for p in range(2):
        pltpu.sync_copy(asv_all.at[p], asv_t)
        pltpu.sync_copy(adv_all.at[p], adv_t)
        h_halves = (h1a, h1b) if p == 0 else (h2a, h2b)
        srcs = srcs1 if p == 0 else srcs2
        dsts = dsts1 if p == 0 else dsts2
        def hop_body(hop, hcarry):
            s_i = p * 3 + hop
            pltpu.sync_copy(srcs.at[hop, wid], src2d)
            pltpu.sync_copy(dsts.at[hop, wid], dst2d)

            def _ex(j, carry):
                for k in range(8):
                    sl = pl.ds(k * 16, 16)
                    s16 = src2d[j, sl]
                    d16 = dst2d[j, sl]
                    av = plsc.load_gather(asv_t, [s16])
                    dv = plsc.load_gather(adv_t, [d16])
                    tt = av + dv
                    e = jnp.where(tt >= 0, tt, 0.2 * tt)
                    ex2d[j, sl] = jnp.exp(e)
                return carry
            lax.fori_loop(0, NCHUNK, _ex, 0)

            @pl.when(hop > 0)
            def _():
                _wait_copyout()
            _zero_acc_slice()
            pltpu.sync_copy(zs, s_sh.at[pl.ds(base, ROWS_PER_TILE)])
            plsc.subcore_barrier()

            for half in range(2):
                h_hbm = h_halves[half]
                bufs = (rows0, rows1, rows2)
                gsem = tuple(gsem_arr.at[u] for u in range(3))
                ssem = tuple(ssem_arr.at[u] for u in range(3))

                def fire_gather(m, u):
                    pltpu.async_copy(h_hbm.at[src2d.at[m]], bufs[u], gsem[u])

                def wait_gather(m, u):
                    pltpu.make_async_copy(h_hbm.at[src2d.at[m]], bufs[u],
                                          gsem[u]).wait()

                def fire_scatter(m, u):
                    pltpu.async_copy(bufs[u], acc_sh.at[dst2d.at[m]],
                                     ssem[u], add=True)
                    if half == 0:
                        pltpu.async_copy(ex2d.at[m], s_sh.at[dst2d.at[m]],
                                         sem_ss, add=True)

                def wait_scatter(m, u):
                    pltpu.make_async_copy(bufs[u], acc_sh.at[dst2d.at[m]],
                                          ssem[u]).wait()
                    if half == 0:
                        pltpu.make_async_copy(ex2d.at[m],
                                              s_sh.at[dst2d.at[m]],
                                              sem_ss).wait()

                def do_slot(j, u, wait_prev=True, fire_next=True):
                    wait_gather(j, u)
                    scale_chunk(bufs[u], j)
                    fire_scatter(j, u)
                    if wait_prev:
                        wait_scatter(j - 1, (u + 2) % 3)
                    if fire_next:
                        fire_gather(j + 2, (u + 2) % 3)

                fire_gather(0, 0)
                fire_gather(1, 1)
                do_slot(0, 0, wait_prev=False)
                do_slot(1, 1)

                def _mid(jj, carry):
                    j = 2 + jj * 3
                    do_slot(j, 2)
                    do_slot(j + 1, 0)
                    do_slot(j + 2, 1)
                    return carry
                lax.fori_loop(0, (NCHUNK - 5) // 3, _mid, 0)

                do_slot(NCHUNK - 3, 2)
                do_slot(NCHUNK - 2, 0, fire_next=False)
                do_slot(NCHUNK - 1, 1, fire_next=False)
                wait_scatter(NCHUNK - 1, 1)
                plsc.subcore_barrier()
                if half == 0:
                    pltpu.sync_copy(
                        acc_sh.at[pl.ds(base, ROWS_PER_TILE), :],
                        out_part.at[c, s_i, half,
                                    pl.ds(base, ROWS_PER_TILE), :])
                    _zero_acc_slice()
                    plsc.subcore_barrier()
                else:
                    pltpu.async_copy(
                        acc_sh.at[pl.ds(base, ROWS_PER_TILE), :],
                        out_part.at[c, s_i, half,
                                    pl.ds(base, ROWS_PER_TILE), :],
                        sem_out)
                    pltpu.async_copy(
                        s_sh.at[pl.ds(base, ROWS_PER_TILE)],
                        s_part.at[c, s_i, pl.ds(base, ROWS_PER_TILE)],
                        sem_out)
            return hcarry
        lax.fori_loop(0, 3, hop_body, 0)
        _wait_copyout()


@functools.cache
def _build_edge_kernel():
    return functools.partial(
        pl.kernel,
        out_type=[
            jax.ShapeDtypeStruct((NC, 6, 2, N_PAD, HH), jnp.float32),
            jax.ShapeDtypeStruct((NC, 6, N_PAD), jnp.float32),
        ],
        mesh=plsc.VectorSubcoreMesh(core_axis_name="c", subcore_axis_name="s",
                                    num_cores=NC, num_subcores=NS),
        compiler_params=pltpu.CompilerParams(needs_layout_passes=False,
                                             use_tc_tiling_on_sc=False),
        scratch_types=[
            pltpu.VMEM((N_PAD,), jnp.float32),
            pltpu.VMEM((N_PAD,), jnp.float32),
            pltpu.VMEM((NCHUNK, 128), jnp.int32),
            pltpu.VMEM((NCHUNK, 128), jnp.int32),
            pltpu.VMEM((NCHUNK, 128), jnp.float32),
            pltpu.VMEM((128, HH), jnp.float32),
            pltpu.VMEM((128, HH), jnp.float32),
            pltpu.VMEM((128, HH), jnp.float32),
            pltpu.VMEM((128, HH), jnp.float32),
            pltpu.VMEM((ROWS_PER_TILE,), jnp.float32),
            pltpu.VMEM_SHARED((N_PAD, HH), jnp.float32),
            pltpu.VMEM_SHARED((N_PAD,), jnp.float32),
            pltpu.SemaphoreType.DMA((3,)),
            pltpu.SemaphoreType.DMA((3,)),
            pltpu.SemaphoreType.DMA,
            pltpu.SemaphoreType.DMA,
        ],
    )(_edge_body)


def _pad_edges(ei):
    pad = EPAD - E
    ar = jnp.arange(pad, dtype=jnp.int32)
    src = jnp.concatenate([ei[0], (ar * 97) % N])
    dst = jnp.concatenate([ei[1], N + (ar % (N_PAD - N))])
    return (src.reshape(NW, NCHUNK, 128), dst.reshape(NW, NCHUNK, 128))


_RB = 1024


def _combine_body(op_ref, sp_ref, b1_ref, b2_ref, p1_ref, p2_ref,
                  pooled1_ref, pooled2_ref, cnt1_ref, cnt2_ref):
    i = pl.program_id(0)

    @pl.when(i == 0)
    def _():
        pooled1_ref[...] = jnp.zeros_like(pooled1_ref)
        pooled2_ref[...] = jnp.zeros_like(pooled2_ref)
        cnt1_ref[...] = jnp.zeros_like(cnt1_ref)
        cnt2_ref[...] = jnp.zeros_like(cnt2_ref)

    sp = sp_ref[...]
    ones = jnp.ones((_RB, H), jnp.float32)
    dn = (((0,), (0,)), ((), ()))
    for p in range(2):
        b_ref = b1_ref if p == 0 else b2_ref
        p_blk = (p1_ref if p == 0 else p2_ref)[...]
        cnt = lax.dot_general(p_blk, ones, dn,
                              preferred_element_type=jnp.float32)
        pooled_halves = []
        for half in range(2):
            xacc = jnp.zeros((_RB, HH), jnp.float32)
            for hop in range(3):
                s_i = p * 3 + hop
                raw = op_ref[0, s_i, half] + op_ref[1, s_i, half]
                s = sp[:, s_i:s_i + 1] + sp[:, 6 + s_i:7 + s_i]
                xh = raw / (s + 1e-16) + b_ref[:, half * HH:(half + 1) * HH]
                xacc = xacc + _leaky(xh, 0.01)
            pooled_halves.append(
                lax.dot_general(p_blk, xacc, dn,
                                preferred_element_type=jnp.float32))
        pooled_ref = pooled1_ref if p == 0 else pooled2_ref
        cnt_ref = cnt1_ref if p == 0 else cnt2_ref
        pooled_ref[:, 0:HH] += pooled_halves[0]
        pooled_ref[:, HH:H] += pooled_halves[1]
        cnt_ref[...] += cnt


def _combine(out_part, s_part, b1, b2, P1, P2):
    sT = s_part.reshape(NC * 6, N_PAD).T
    return pl.pallas_call(
        _combine_body,
        grid=(N_PAD // _RB,),
        in_specs=[
            pl.BlockSpec((NC, 6, 2, _RB, HH), lambda i: (0, 0, 0, i, 0)),
            pl.BlockSpec((_RB, NC * 6), lambda i: (i, 0)),
            pl.BlockSpec((1, H), lambda i: (0, 0)),
            pl.BlockSpec((1, H), lambda i: (0, 0)),
            pl.BlockSpec((_RB, G), lambda i: (i, 0)),
            pl.BlockSpec((_RB, G), lambda i: (i, 0)),
        ],
        out_specs=[
            pl.BlockSpec((G, H), lambda i: (0, 0)),
            pl.BlockSpec((G, H), lambda i: (0, 0)),
            pl.BlockSpec((G, H), lambda i: (0, 0)),
            pl.BlockSpec((G, H), lambda i: (0, 0)),
        ],
        out_shape=[jax.ShapeDtypeStruct((G, H), jnp.float32)] * 4,
    )(out_part, sT, b1.reshape(1, H), b2.reshape(1, H), P1, P2)


def _head_body(pooled1_ref, pooled2_ref, cnt1_ref, cnt2_ref,
               fc1p_w_ref, fc1p_b_ref, fc2p_w_ref, fc2p_b_ref,
               fcc1_w_ref, fcc1_b_ref, fcc2_w_ref, fcc2_b_ref,
               out_w_ref, out_b_ref, out_ref):
    x1 = pooled1_ref[...] / jnp.maximum(cnt1_ref[...], 1.0)
    x2 = pooled2_ref[...] / jnp.maximum(cnt2_ref[...], 1.0)
    z1 = _leaky(jnp.dot(x1, fc1p_w_ref[...],
                        preferred_element_type=jnp.float32)
                + fc1p_b_ref[...], 0.01)
    z2 = _leaky(jnp.dot(x2, fc2p_w_ref[...],
                        preferred_element_type=jnp.float32)
                + fc2p_b_ref[...], 0.01)
    xc = (jnp.dot(z1, fcc1_w_ref[0:H, :], preferred_element_type=jnp.float32)
          + jnp.dot(z2, fcc1_w_ref[H:2 * H, :],
                    preferred_element_type=jnp.float32)
          + fcc1_b_ref[...])
    xc = _leaky(xc, 0.01)
    xc = _leaky(jnp.dot(xc, fcc2_w_ref[...],
                        preferred_element_type=jnp.float32)
                + fcc2_b_ref[...], 0.01)
    z = jnp.dot(xc, out_w_ref[...],
                preferred_element_type=jnp.float32) + out_b_ref[...]
    out_ref[...] = 1.0 / (1.0 + jnp.exp(-z))


def _head(pooled1, pooled2, cnt1, cnt2, fc1p_w, fc1p_b, fc2p_w, fc2p_b,
          fcc1_w, fcc1_b, fcc2_w, fcc2_b, out_w, out_b):
    return pl.pallas_call(
        _head_body,
        out_shape=jax.ShapeDtypeStruct((G, 1), jnp.float32),
    )(pooled1, pooled2, cnt1, cnt2,
      fc1p_w, fc1p_b.reshape(1, 128), fc2p_w, fc2p_b.reshape(1, 128),
      fcc1_w, fcc1_b.reshape(1, 256), fcc2_w, fcc2_b.reshape(1, 64),
      out_w, out_b.reshape(1, 1))


def kernel(pro1_x, pro1_edge_index, pro1_two_hop_edge_index, pro1_three_hop_edge_index, pro1_batch, pro2_x, pro2_edge_index, pro2_two_hop_edge_index, pro2_three_hop_edge_index, pro2_batch, W1, a_src1, a_dst1, b1, W2, a_src2, a_dst2, b2, fc1p_w, fc1p_b, fc2p_w, fc2p_b, fcc1_w, fcc1_b, fcc2_w, fcc2_b, out_w, out_b):
    h1, asv1, adv1 = _projections(pro1_x, W1, a_src1, a_dst1)
    h2, asv2, adv2 = _projections(pro2_x, W2, a_src2, a_dst2)

    e1 = [_pad_edges(e) for e in
          (pro1_edge_index, pro1_two_hop_edge_index, pro1_three_hop_edge_index)]
    e2 = [_pad_edges(e) for e in
          (pro2_edge_index, pro2_two_hop_edge_index, pro2_three_hop_edge_index)]
    srcs1 = jnp.stack([s for s, _ in e1])
    dsts1 = jnp.stack([d for _, d in e1])
    srcs2 = jnp.stack([s for s, _ in e2])
    dsts2 = jnp.stack([d for _, d in e2])
    zpad = jnp.zeros((N_PAD - N,), jnp.float32)
    asv_all = jnp.stack([jnp.concatenate([asv1, zpad]),
                         jnp.concatenate([asv2, zpad])])
    adv_all = jnp.stack([jnp.concatenate([adv1, zpad]),
                         jnp.concatenate([adv2, zpad])])

    out_part, s_part = _build_edge_kernel()(
        srcs1, dsts1, srcs2, dsts2, asv_all, adv_all,
        h1[:, :HH], h1[:, HH:], h2[:, :HH], h2[:, HH:])

    ar = jnp.arange(G, dtype=jnp.int32)
    b1p = jnp.concatenate([pro1_batch, jnp.full((N_PAD - N,), G,
                                                jnp.int32)])
    b2p = jnp.concatenate([pro2_batch, jnp.full((N_PAD - N,), G,
                                                jnp.int32)])
    P1 = (b1p[:, None] == ar[None, :]).astype(jnp.float32)
    P2 = (b2p[:, None] == ar[None, :]).astype(jnp.float32)

    pooled1, pooled2, cnt1, cnt2 = _combine(out_part, s_part, b1, b2, P1, P2)
    return _head(pooled1, pooled2, cnt1, cnt2,
                 fc1p_w, fc1p_b, fc2p_w, fc2p_b,
                 fcc1_w, fcc1_b, fcc2_w, fcc2_b, out_w, out_b)
